# baseline (speedup 1.0000x reference)
import functools

import jax
import jax.numpy as jnp
from jax import lax
from jax.experimental import pallas as pl
from jax.experimental.pallas import tpu as pltpu
from jax.experimental.pallas import tpu_sc as plsc

N = 10000
E = 320000
D = 128
EPS = 1e-5

NC, NS = 2, 16
NW = NC * NS
ECH = 128
NCH = 80
EPT = ECH * NCH
EPAD = NW * EPT - E
NP = 10240
RPT = NP // NS

_mesh = plsc.VectorSubcoreMesh(
    core_axis_name="c", subcore_axis_name="s", num_cores=NC, num_subcores=NS
)


@functools.partial(
    pl.kernel,
    out_type=jax.ShapeDtypeStruct((NC * NP,), jnp.float32),
    mesh=_mesh,
    scratch_types=[
        pltpu.VMEM((NCH, ECH), jnp.int32),
        pltpu.VMEM((ECH,), jnp.float32),
        pltpu.VMEM((RPT,), jnp.float32),
        pltpu.VMEM_SHARED((NP,), jnp.float32),
        pltpu.SemaphoreType.DMA,
    ],
)
def _sc_degree(col3d, degp, colbuf, ones, zbuf, deg_sh, hsem):
    c = lax.axis_index("c")
    s = lax.axis_index("s")
    wid = c * NS + s

    def fill_ones(i, _):
        ones[pl.ds(i * 16, 16)] = jnp.ones((16,), jnp.float32)
        return 0

    lax.fori_loop(0, ECH // 16, fill_ones, 0)

    def fill_zero(i, _):
        zbuf[pl.ds(i * 16, 16)] = jnp.zeros((16,), jnp.float32)
        return 0

    lax.fori_loop(0, RPT // 16, fill_zero, 0)

    pltpu.sync_copy(zbuf, deg_sh.at[pl.ds(s * RPT, RPT)])
    plsc.subcore_barrier()

    pltpu.sync_copy(col3d.at[wid], colbuf)

    descs = [
        pltpu.async_copy(ones, deg_sh.at[colbuf.at[j]], hsem, add=True)
        for j in range(NCH)
    ]
    for d in descs:
        d.wait()

    plsc.subcore_barrier()
    pltpu.sync_copy(deg_sh.at[pl.ds(s * RPT, RPT)],
                    degp.at[pl.ds(c * NP + s * RPT, RPT)])


@functools.partial(
    pl.kernel,
    out_type=jax.ShapeDtypeStruct((NC, NS, RPT, D), jnp.float32),
    mesh=_mesh,
    scratch_types=[
        pltpu.VMEM((3, ECH), jnp.int32),
        pltpu.VMEM((3, ECH), jnp.int32),
        pltpu.VMEM((2, ECH, D), jnp.float32),
        pltpu.VMEM_SHARED((NP, D), jnp.float32),
        pltpu.SemaphoreType.DMA,
        pltpu.SemaphoreType.DMA,
        pltpu.SemaphoreType.DMA,
    ],
)
def _sc_aggregate(g_hbm, row3d, col3d, zeros_hbm, accp, rbuf, cbuf, gbuf,
                  acc_sh, gsem, isem, ssem):
    c = lax.axis_index("c")
    s = lax.axis_index("s")
    wid = c * NS + s

    zero_d = [
        pltpu.async_copy(zeros_hbm, acc_sh.at[pl.ds(s * RPT + t * ECH, ECH)],
                         ssem)
        for t in range(RPT // ECH)
    ]

    idx_d = [None] * NCH
    gat_d = [None] * NCH
    sca_d = [None] * NCH

    def load_idx(j):
        b = j % 3
        r = pltpu.async_copy(row3d.at[wid, j], rbuf.at[b], isem)
        cc = pltpu.async_copy(col3d.at[wid, j], cbuf.at[b], isem)
        idx_d[j] = (r, cc)

    def start_gather(j):
        gat_d[j] = pltpu.async_copy(g_hbm.at[rbuf.at[j % 3]],
                                    gbuf.at[j % 2], gsem)

    load_idx(0)
    load_idx(1)
    load_idx(2)
    idx_d[0][0].wait()
    idx_d[0][1].wait()
    start_gather(0)
    idx_d[1][0].wait()
    idx_d[1][1].wait()
    start_gather(1)
    for d in zero_d:
        d.wait()
    plsc.subcore_barrier()

    for j in range(NCH):
        gat_d[j].wait()
        sca_d[j] = pltpu.async_copy(gbuf.at[j % 2], acc_sh.at[cbuf.at[j % 3]],
                                    ssem, add=True)
        if j + 2 < NCH:
            idx_d[j + 2][0].wait()
            idx_d[j + 2][1].wait()
            sca_d[j].wait()
            start_gather(j + 2)
            if j + 3 < NCH:
                load_idx(j + 3)
        else:
            sca_d[j].wait()

    plsc.subcore_barrier()
    pltpu.sync_copy(acc_sh.at[pl.ds(s * RPT, RPT)], accp.at[c, s])


def _tc_fused_body(x_ref, gamma_ref, beta_ref, w_ref, deg_ref, g_ref):
    x = x_ref[...]
    mean = jnp.mean(x, axis=-1, keepdims=True)
    xc = x - mean
    var = jnp.mean(xc * xc, axis=-1, keepdims=True)
    h = xc * lax.rsqrt(var + EPS) * gamma_ref[...] + beta_ref[...]
    h = jnp.maximum(h, 0.0)
    h = jnp.dot(h, w_ref[...], preferred_element_type=jnp.float32)
    g_ref[...] = h * lax.rsqrt(deg_ref[...] + 1.0)


def _tc_fused(x, gamma2, beta2, W, degcol):
    blk = 2000
    grid = N // blk
    return pl.pallas_call(
        _tc_fused_body,
        out_shape=jax.ShapeDtypeStruct((N, D), jnp.float32),
        grid=(grid,),
        in_specs=[
            pl.BlockSpec((blk, D), lambda i: (i, 0)),
            pl.BlockSpec((1, D), lambda i: (0, 0)),
            pl.BlockSpec((1, D), lambda i: (0, 0)),
            pl.BlockSpec((D, D), lambda i: (0, 0)),
            pl.BlockSpec((blk, 1), lambda i: (i, 0)),
        ],
        out_specs=pl.BlockSpec((blk, D), lambda i: (i, 0)),
    )(x, gamma2, beta2, W, degcol)


def _tc_out_body(accp_ref, g_ref, deg_ref, b_ref, out_ref):
    acc = accp_ref[0] + accp_ref[1]
    dinv = lax.rsqrt(deg_ref[...] + 1.0)
    out_ref[...] = dinv * (acc + g_ref[...]) + b_ref[...]


def _tc_out(accp, g, degcol, b2):
    blk = 2000
    grid = N // blk
    return pl.pallas_call(
        _tc_out_body,
        out_shape=jax.ShapeDtypeStruct((N, D), jnp.float32),
        grid=(grid,),
        in_specs=[
            pl.BlockSpec((NC, blk, D), lambda i: (0, i, 0)),
            pl.BlockSpec((blk, D), lambda i: (i, 0)),
            pl.BlockSpec((blk, 1), lambda i: (i, 0)),
            pl.BlockSpec((1, D), lambda i: (0, 0)),
        ],
        out_specs=pl.BlockSpec((blk, D), lambda i: (i, 0)),
    )(accp, g, degcol, b2)


def kernel(x, edge_index, gamma, beta, W, b):
    pad_row = (jnp.arange(EPAD, dtype=jnp.int32) * 131) % N
    pad_col = N + (jnp.arange(EPAD, dtype=jnp.int32) % (NP - N))
    row3d = jnp.concatenate([edge_index[0], pad_row]).reshape(NW, NCH, ECH)
    col3d = jnp.concatenate([edge_index[1], pad_col]).reshape(NW, NCH, ECH)

    degp = _sc_degree(col3d).reshape(NC, NP)
    degcol = (degp[0, :N] + degp[1, :N])[:, None]
    g = _tc_fused(x, gamma.reshape(1, D), beta.reshape(1, D), W, degcol)
    zeros_hbm = jnp.zeros((ECH, D), jnp.float32)
    accp = _sc_aggregate(g, row3d, col3d, zeros_hbm).reshape(NC, NP, D)
    return _tc_out(accp, g, degcol, b.reshape(1, D))


# speedup vs baseline: 1.0861x; 1.0861x over previous
import functools

import jax
import jax.numpy as jnp
from jax import lax
from jax.experimental import pallas as pl
from jax.experimental.pallas import tpu as pltpu
from jax.experimental.pallas import tpu_sc as plsc

N = 10000
E = 320000
D = 128
EPS = 1e-5

NC, NS = 2, 16
NW = NC * NS
ECH = 128
NCH = 80
EPT = ECH * NCH
EPAD = NW * EPT - E
NP = 10240
RPT = NP // NS

_mesh = plsc.VectorSubcoreMesh(
    core_axis_name="c", subcore_axis_name="s", num_cores=NC, num_subcores=NS
)


@functools.partial(
    pl.kernel,
    out_type=jax.ShapeDtypeStruct((NC * NP,), jnp.float32),
    mesh=_mesh,
    scratch_types=[
        pltpu.VMEM((NCH, ECH), jnp.int32),
        pltpu.VMEM((ECH,), jnp.float32),
        pltpu.VMEM((RPT,), jnp.float32),
        pltpu.VMEM_SHARED((NP,), jnp.float32),
        pltpu.SemaphoreType.DMA,
    ],
)
def _sc_degree(col3d, degp, colbuf, ones, zbuf, deg_sh, hsem):
    c = lax.axis_index("c")
    s = lax.axis_index("s")
    wid = c * NS + s

    def fill_ones(i, _):
        ones[pl.ds(i * 16, 16)] = jnp.ones((16,), jnp.float32)
        return 0

    lax.fori_loop(0, ECH // 16, fill_ones, 0)

    def fill_zero(i, _):
        zbuf[pl.ds(i * 16, 16)] = jnp.zeros((16,), jnp.float32)
        return 0

    lax.fori_loop(0, RPT // 16, fill_zero, 0)

    pltpu.sync_copy(zbuf, deg_sh.at[pl.ds(s * RPT, RPT)])
    plsc.subcore_barrier()

    pltpu.sync_copy(col3d.at[wid], colbuf)

    descs = [
        pltpu.async_copy(ones, deg_sh.at[colbuf.at[j]], hsem, add=True)
        for j in range(NCH)
    ]
    for d in descs:
        d.wait()

    plsc.subcore_barrier()
    pltpu.sync_copy(deg_sh.at[pl.ds(s * RPT, RPT)],
                    degp.at[pl.ds(c * NP + s * RPT, RPT)])


@functools.partial(
    pl.kernel,
    out_type=jax.ShapeDtypeStruct((NC, NS, RPT, D), jnp.float32),
    mesh=_mesh,
    scratch_types=[
        pltpu.VMEM((3, ECH), jnp.int32),
        pltpu.VMEM((3, ECH), jnp.int32),
        pltpu.VMEM((2, ECH, D), jnp.float32),
        pltpu.VMEM_SHARED((NP, D), jnp.float32),
        pltpu.SemaphoreType.DMA,
        pltpu.SemaphoreType.DMA,
        pltpu.SemaphoreType.DMA,
    ],
)
def _sc_aggregate(g_hbm, row3d, col3d, accp, rbuf, cbuf, gbuf,
                  acc_sh, gsem, isem, ssem):
    c = lax.axis_index("c")
    s = lax.axis_index("s")
    wid = c * NS + s

    def fill_zero(i, _):
        for k in range(8):
            gbuf[0, i, pl.ds(k * 16, 16)] = jnp.zeros((16,), jnp.float32)
        return 0

    lax.fori_loop(0, ECH, fill_zero, 0)
    zero_d = [
        pltpu.async_copy(gbuf.at[0], acc_sh.at[pl.ds(s * RPT + t * ECH, ECH)],
                         ssem)
        for t in range(RPT // ECH)
    ]

    idx_d = [None] * NCH
    gat_d = [None] * NCH
    sca_d = [None] * NCH

    def load_idx(j):
        b = j % 3
        r = pltpu.async_copy(row3d.at[wid, j], rbuf.at[b], isem)
        cc = pltpu.async_copy(col3d.at[wid, j], cbuf.at[b], isem)
        idx_d[j] = (r, cc)

    def start_gather(j):
        gat_d[j] = pltpu.async_copy(g_hbm.at[rbuf.at[j % 3]],
                                    gbuf.at[j % 2], gsem)

    load_idx(0)
    load_idx(1)
    load_idx(2)
    for d in zero_d:
        d.wait()
    idx_d[0][0].wait()
    idx_d[0][1].wait()
    start_gather(0)
    idx_d[1][0].wait()
    idx_d[1][1].wait()
    start_gather(1)
    plsc.subcore_barrier()

    for j in range(NCH):
        gat_d[j].wait()
        sca_d[j] = pltpu.async_copy(gbuf.at[j % 2], acc_sh.at[cbuf.at[j % 3]],
                                    ssem, add=True)
        if j + 2 < NCH:
            idx_d[j + 2][0].wait()
            idx_d[j + 2][1].wait()
            sca_d[j].wait()
            start_gather(j + 2)
            if j + 3 < NCH:
                load_idx(j + 3)
        else:
            sca_d[j].wait()

    plsc.subcore_barrier()
    pltpu.sync_copy(acc_sh.at[pl.ds(s * RPT, RPT)], accp.at[c, s])


def _tc_fused_body(x_ref, gamma_ref, beta_ref, w_ref, deg_ref, g_ref):
    x = x_ref[...]
    mean = jnp.mean(x, axis=-1, keepdims=True)
    xc = x - mean
    var = jnp.mean(xc * xc, axis=-1, keepdims=True)
    h = xc * lax.rsqrt(var + EPS) * gamma_ref[...] + beta_ref[...]
    h = jnp.maximum(h, 0.0)
    h = jnp.dot(h, w_ref[...], preferred_element_type=jnp.float32)
    g_ref[...] = h * lax.rsqrt(deg_ref[...] + 1.0)


def _tc_fused(x, gamma2, beta2, W, degcol):
    blk = 2000
    grid = N // blk
    return pl.pallas_call(
        _tc_fused_body,
        out_shape=jax.ShapeDtypeStruct((N, D), jnp.float32),
        grid=(grid,),
        in_specs=[
            pl.BlockSpec((blk, D), lambda i: (i, 0)),
            pl.BlockSpec((1, D), lambda i: (0, 0)),
            pl.BlockSpec((1, D), lambda i: (0, 0)),
            pl.BlockSpec((D, D), lambda i: (0, 0)),
            pl.BlockSpec((blk, 1), lambda i: (i, 0)),
        ],
        out_specs=pl.BlockSpec((blk, D), lambda i: (i, 0)),
    )(x, gamma2, beta2, W, degcol)


def _tc_out_body(accp_ref, g_ref, deg_ref, b_ref, out_ref):
    acc = accp_ref[0] + accp_ref[1]
    dinv = lax.rsqrt(deg_ref[...] + 1.0)
    out_ref[...] = dinv * (acc + g_ref[...]) + b_ref[...]


def _tc_out(accp, g, degcol, b2):
    blk = 2000
    grid = N // blk
    return pl.pallas_call(
        _tc_out_body,
        out_shape=jax.ShapeDtypeStruct((N, D), jnp.float32),
        grid=(grid,),
        in_specs=[
            pl.BlockSpec((NC, blk, D), lambda i: (0, i, 0)),
            pl.BlockSpec((blk, D), lambda i: (i, 0)),
            pl.BlockSpec((blk, 1), lambda i: (i, 0)),
            pl.BlockSpec((1, D), lambda i: (0, 0)),
        ],
        out_specs=pl.BlockSpec((blk, D), lambda i: (i, 0)),
    )(accp, g, degcol, b2)


def kernel(x, edge_index, gamma, beta, W, b):
    pad_row = (jnp.arange(EPAD, dtype=jnp.int32) * 131) % N
    pad_col = N + (jnp.arange(EPAD, dtype=jnp.int32) % (NP - N))
    row3d = jnp.concatenate([edge_index[0], pad_row]).reshape(NW, NCH, ECH)
    col3d = jnp.concatenate([edge_index[1], pad_col]).reshape(NW, NCH, ECH)

    degp = _sc_degree(col3d).reshape(NC, NP)
    degcol = (degp[0, :N] + degp[1, :N])[:, None]
    g = _tc_fused(x, gamma.reshape(1, D), beta.reshape(1, D), W, degcol)
    accp = _sc_aggregate(g, row3d, col3d).reshape(NC, NP, D)
    return _tc_out(accp, g, degcol, b.reshape(1, D))


# speedup vs baseline: 1.0909x; 1.0044x over previous
import functools

import jax
import jax.numpy as jnp
from jax import lax
from jax.experimental import pallas as pl
from jax.experimental.pallas import tpu as pltpu
from jax.experimental.pallas import tpu_sc as plsc

N = 10000
E = 320000
D = 128
EPS = 1e-5

NC, NS = 2, 16
NW = NC * NS
ECH = 128
NCH = 80
EPT = ECH * NCH
EPAD = NW * EPT - E
NP = 10240
RPT = NP // NS

_mesh = plsc.VectorSubcoreMesh(
    core_axis_name="c", subcore_axis_name="s", num_cores=NC, num_subcores=NS
)


@functools.partial(
    pl.kernel,
    out_type=jax.ShapeDtypeStruct((NC * NP,), jnp.float32),
    mesh=_mesh,
    scratch_types=[
        pltpu.VMEM((NCH, ECH), jnp.int32),
        pltpu.VMEM((ECH,), jnp.float32),
        pltpu.VMEM((RPT,), jnp.float32),
        pltpu.VMEM_SHARED((NP,), jnp.float32),
        pltpu.SemaphoreType.DMA,
    ],
)
def _sc_degree(col3d, degp, colbuf, ones, zbuf, deg_sh, hsem):
    c = lax.axis_index("c")
    s = lax.axis_index("s")
    wid = c * NS + s

    def fill_ones(i, _):
        ones[pl.ds(i * 16, 16)] = jnp.ones((16,), jnp.float32)
        return 0

    lax.fori_loop(0, ECH // 16, fill_ones, 0)

    def fill_zero(i, _):
        zbuf[pl.ds(i * 16, 16)] = jnp.zeros((16,), jnp.float32)
        return 0

    lax.fori_loop(0, RPT // 16, fill_zero, 0)

    pltpu.sync_copy(zbuf, deg_sh.at[pl.ds(s * RPT, RPT)])
    plsc.subcore_barrier()

    pltpu.sync_copy(col3d.at[wid], colbuf)

    descs = [
        pltpu.async_copy(ones, deg_sh.at[colbuf.at[j]], hsem, add=True)
        for j in range(NCH)
    ]
    for d in descs:
        d.wait()

    plsc.subcore_barrier()
    pltpu.sync_copy(deg_sh.at[pl.ds(s * RPT, RPT)],
                    degp.at[pl.ds(c * NP + s * RPT, RPT)])


@functools.partial(
    pl.kernel,
    out_type=jax.ShapeDtypeStruct((NC, NS, RPT, D), jnp.float32),
    mesh=_mesh,
    scratch_types=[
        pltpu.VMEM((3, ECH), jnp.int32),
        pltpu.VMEM((3, ECH), jnp.int32),
        pltpu.VMEM((2, ECH, D), jnp.float32),
        pltpu.VMEM_SHARED((NP, D), jnp.float32),
        pltpu.SemaphoreType.DMA,
        pltpu.SemaphoreType.DMA,
        pltpu.SemaphoreType.DMA,
    ],
)
def _sc_aggregate(g_hbm, row3d, col3d, accp, rbuf, cbuf, gbuf,
                  acc_sh, gsem, isem, ssem):
    c = lax.axis_index("c")
    s = lax.axis_index("s")
    wid = c * NS + s

    def fill_zero(i, _):
        for k in range(8):
            gbuf[0, i, pl.ds(k * 16, 16)] = jnp.zeros((16,), jnp.float32)
        return 0

    lax.fori_loop(0, ECH, fill_zero, 0)
    zero_d = [
        pltpu.async_copy(gbuf.at[0], acc_sh.at[pl.ds(s * RPT + t * ECH, ECH)],
                         ssem)
        for t in range(RPT // ECH)
    ]

    idx_d = [None] * NCH
    gat_d = [None] * NCH
    sca_d = [None] * NCH

    def load_idx(j):
        b = j % 3
        r = pltpu.async_copy(row3d.at[wid, j], rbuf.at[b], isem)
        cc = pltpu.async_copy(col3d.at[wid, j], cbuf.at[b], isem)
        idx_d[j] = (r, cc)

    def start_gather(j):
        gat_d[j] = pltpu.async_copy(g_hbm.at[rbuf.at[j % 3]],
                                    gbuf.at[j % 2], gsem)

    load_idx(0)
    load_idx(1)
    load_idx(2)
    for d in zero_d:
        d.wait()
    idx_d[0][0].wait()
    idx_d[0][1].wait()
    start_gather(0)
    idx_d[1][0].wait()
    idx_d[1][1].wait()
    start_gather(1)
    plsc.subcore_barrier()

    for j in range(NCH):
        gat_d[j].wait()
        sca_d[j] = pltpu.async_copy(gbuf.at[j % 2], acc_sh.at[cbuf.at[j % 3]],
                                    ssem, add=True)
        if j + 2 < NCH:
            idx_d[j + 2][0].wait()
            idx_d[j + 2][1].wait()
            sca_d[j].wait()
            start_gather(j + 2)
            if j + 3 < NCH:
                load_idx(j + 3)
        else:
            sca_d[j].wait()

    plsc.subcore_barrier()
    pltpu.sync_copy(acc_sh.at[pl.ds(s * RPT, RPT)], accp.at[c, s])


def _tc_lnmm_body(x_ref, gamma_ref, beta_ref, w_ref, h_ref):
    x = x_ref[...]
    mean = jnp.mean(x, axis=-1, keepdims=True)
    xc = x - mean
    var = jnp.mean(xc * xc, axis=-1, keepdims=True)
    h = xc * lax.rsqrt(var + EPS) * gamma_ref[...] + beta_ref[...]
    h = jnp.maximum(h, 0.0)
    h_ref[...] = jnp.dot(h, w_ref[...], preferred_element_type=jnp.float32)


def _tc_lnmm(x, gamma2, beta2, W):
    blk = 2000
    grid = N // blk
    return pl.pallas_call(
        _tc_lnmm_body,
        out_shape=jax.ShapeDtypeStruct((N, D), jnp.float32),
        grid=(grid,),
        in_specs=[
            pl.BlockSpec((blk, D), lambda i: (i, 0)),
            pl.BlockSpec((1, D), lambda i: (0, 0)),
            pl.BlockSpec((1, D), lambda i: (0, 0)),
            pl.BlockSpec((D, D), lambda i: (0, 0)),
        ],
        out_specs=pl.BlockSpec((blk, D), lambda i: (i, 0)),
    )(x, gamma2, beta2, W)


def _tc_scale_body(h_ref, deg_ref, g_ref):
    g_ref[...] = h_ref[...] * lax.rsqrt(deg_ref[...] + 1.0)


def _tc_scale(h, degcol):
    blk = 2000
    grid = N // blk
    return pl.pallas_call(
        _tc_scale_body,
        out_shape=jax.ShapeDtypeStruct((N, D), jnp.float32),
        grid=(grid,),
        in_specs=[
            pl.BlockSpec((blk, D), lambda i: (i, 0)),
            pl.BlockSpec((blk, 1), lambda i: (i, 0)),
        ],
        out_specs=pl.BlockSpec((blk, D), lambda i: (i, 0)),
    )(h, degcol)


def _tc_out_body(accp_ref, g_ref, deg_ref, b_ref, out_ref):
    acc = accp_ref[0] + accp_ref[1]
    dinv = lax.rsqrt(deg_ref[...] + 1.0)
    out_ref[...] = dinv * (acc + g_ref[...]) + b_ref[...]


def _tc_out(accp, g, degcol, b2):
    blk = 2000
    grid = N // blk
    return pl.pallas_call(
        _tc_out_body,
        out_shape=jax.ShapeDtypeStruct((N, D), jnp.float32),
        grid=(grid,),
        in_specs=[
            pl.BlockSpec((NC, blk, D), lambda i: (0, i, 0)),
            pl.BlockSpec((blk, D), lambda i: (i, 0)),
            pl.BlockSpec((blk, 1), lambda i: (i, 0)),
            pl.BlockSpec((1, D), lambda i: (0, 0)),
        ],
        out_specs=pl.BlockSpec((blk, D), lambda i: (i, 0)),
    )(accp, g, degcol, b2)


def kernel(x, edge_index, gamma, beta, W, b):
    pad_row = (jnp.arange(EPAD, dtype=jnp.int32) * 131) % N
    pad_col = N + (jnp.arange(EPAD, dtype=jnp.int32) % (NP - N))
    row3d = jnp.concatenate([edge_index[0], pad_row]).reshape(NW, NCH, ECH)
    col3d = jnp.concatenate([edge_index[1], pad_col]).reshape(NW, NCH, ECH)

    degp = _sc_degree(col3d).reshape(NC, NP)
    h = _tc_lnmm(x, gamma.reshape(1, D), beta.reshape(1, D), W)
    degcol = (degp[0, :N] + degp[1, :N])[:, None]
    g = _tc_scale(h, degcol)
    accp = _sc_aggregate(g, row3d, col3d).reshape(NC, NP, D)
    return _tc_out(accp, g, degcol, b.reshape(1, D))
